# Initial kernel scaffold; baseline (speedup 1.0000x reference)
import jax
import jax.numpy as jnp
from jax.experimental import pallas as pl


def kernel(graph_embedding, locations_idx, W, b, gamma, beta):
    raise NotImplementedError("write your pallas kernel here")


# speedup vs baseline: 6.7046x; 6.7046x over previous
import functools

import jax
import jax.numpy as jnp
from jax import lax
from jax.experimental import pallas as pl
from jax.experimental.pallas import tpu as pltpu
from jax.experimental.pallas import tpu_sc as plsc

N_NODES = 100000
D_FEAT = 128
B = 1024
L = 200
D_OUT = 256

NC, NS = 2, 16
NW = NC * NS
RPW = B // NW
NCH = 2
CH = L // NCH
NV = D_FEAT // 16


def _sc_gather_max(table, idx2d):
    mesh = plsc.VectorSubcoreMesh(core_axis_name="c", subcore_axis_name="s")

    @functools.partial(
        pl.kernel,
        mesh=mesh,
        out_type=jax.ShapeDtypeStruct((B, D_FEAT), jnp.float32),
        scratch_types=[
            pltpu.VMEM((RPW * NCH, CH), jnp.int32),
            pltpu.VMEM((L, D_FEAT), jnp.float32),
            pltpu.VMEM((RPW, D_FEAT), jnp.float32),
            pltpu.SemaphoreType.DMA,
        ],
    )
    def k(table_hbm, idx_hbm, out_hbm, idx_v, buf_v, res_v, sem):
        wid = lax.axis_index("s") * NC + lax.axis_index("c")
        rows0 = wid * (RPW * NCH)
        pltpu.sync_copy(idx_hbm.at[pl.ds(rows0, RPW * NCH)], idx_v)

        def route_body(rt, carry):
            h0 = pltpu.async_copy(
                table_hbm.at[idx_v.at[NCH * rt]], buf_v.at[pl.ds(0, CH)], sem)
            h1 = pltpu.async_copy(
                table_hbm.at[idx_v.at[NCH * rt + 1]], buf_v.at[pl.ds(CH, CH)],
                sem)
            h0.wait()
            h1.wait()

            def lbody(l, accs):
                return tuple(
                    jnp.maximum(a, buf_v[l, pl.ds(d * 16, 16)])
                    for d, a in enumerate(accs))

            init = tuple(
                jnp.full((16,), -jnp.inf, jnp.float32) for _ in range(NV))
            accs = lax.fori_loop(0, L, lbody, init)
            for d in range(NV):
                res_v[rt, pl.ds(d * 16, 16)] = accs[d]
            return carry

        lax.fori_loop(0, RPW, route_body, 0)
        pltpu.sync_copy(res_v, out_hbm.at[pl.ds(wid * RPW, RPW)])

    return k(table, idx2d)


def _tc_head(x, W, b, gamma, beta):
    def body(x_ref, w_ref, b_ref, g_ref, be_ref, o_ref):
        h = jnp.dot(x_ref[...], w_ref[...],
                    preferred_element_type=jnp.float32)
        h = h + b_ref[...]
        mean = jnp.mean(h, axis=-1, keepdims=True)
        c = h - mean
        var = jnp.mean(c * c, axis=-1, keepdims=True)
        hn = c * lax.rsqrt(var + 1e-5)
        hn = hn * g_ref[...] + be_ref[...]
        o_ref[...] = jnp.maximum(hn, 0.0)

    return pl.pallas_call(
        body,
        out_shape=jax.ShapeDtypeStruct((B, D_OUT), jnp.float32),
    )(x, W, b.reshape(1, D_OUT), gamma.reshape(1, D_OUT),
      beta.reshape(1, D_OUT))


def kernel(graph_embedding, locations_idx, W, b, gamma, beta):
    idx2d = locations_idx.astype(jnp.int32).reshape(B * NCH, CH)
    x = _sc_gather_max(graph_embedding, idx2d)
    return _tc_head(x, W, b, gamma, beta)


# speedup vs baseline: 10.2765x; 1.5328x over previous
import functools

import jax
import jax.numpy as jnp
from jax import lax
from jax.experimental import pallas as pl
from jax.experimental.pallas import tpu as pltpu
from jax.experimental.pallas import tpu_sc as plsc

N_NODES = 100000
D_FEAT = 128
B = 1024
L = 200
D_OUT = 256

NC, NS = 2, 16
NW = NC * NS
RPW = B // NW
NCH = 2
CH = L // NCH
NV = D_FEAT // 16


def _sc_gather_max(table, idx2d):
    mesh = plsc.VectorSubcoreMesh(core_axis_name="c", subcore_axis_name="s")

    @functools.partial(
        pl.kernel,
        mesh=mesh,
        out_type=jax.ShapeDtypeStruct((B, D_FEAT), jnp.float32),
        scratch_types=[
            pltpu.VMEM((RPW * NCH, CH), jnp.int32),
            pltpu.VMEM((L, D_FEAT), jnp.float32),
            pltpu.VMEM((L, D_FEAT), jnp.float32),
            pltpu.VMEM((RPW, D_FEAT), jnp.float32),
            pltpu.SemaphoreType.DMA,
            pltpu.SemaphoreType.DMA,
        ],
    )
    def k(table_hbm, idx_hbm, out_hbm, idx_v, buf_a, buf_b, res_v, sem_a,
          sem_b):
        wid = lax.axis_index("s") * NC + lax.axis_index("c")
        rows0 = wid * (RPW * NCH)
        pltpu.sync_copy(idx_hbm.at[pl.ds(rows0, RPW * NCH)], idx_v)

        def issue(rt, buf, sem):
            pltpu.async_copy(
                table_hbm.at[idx_v.at[NCH * rt]], buf.at[pl.ds(0, CH)], sem)
            pltpu.async_copy(
                table_hbm.at[idx_v.at[NCH * rt + 1]], buf.at[pl.ds(CH, CH)],
                sem)

        def drain(buf, sem):
            pltpu.make_async_copy(
                table_hbm.at[idx_v.at[0]], buf.at[pl.ds(0, CH)], sem).wait()
            pltpu.make_async_copy(
                table_hbm.at[idx_v.at[0]], buf.at[pl.ds(CH, CH)], sem).wait()

        def reduce(rt, buf):
            def lbody(l, accs):
                return tuple(
                    jnp.maximum(a, buf[l, pl.ds(d * 16, 16)])
                    for d, a in enumerate(accs))

            init = tuple(
                jnp.full((16,), -jnp.inf, jnp.float32) for _ in range(NV))
            accs = lax.fori_loop(0, L, lbody, init, unroll=4)
            for d in range(NV):
                res_v[rt, pl.ds(d * 16, 16)] = accs[d]

        issue(0, buf_a, sem_a)

        def pair_body(i, carry):
            issue(2 * i + 1, buf_b, sem_b)
            drain(buf_a, sem_a)
            reduce(2 * i, buf_a)

            @pl.when(i < RPW // 2 - 1)
            def _():
                issue(2 * i + 2, buf_a, sem_a)

            drain(buf_b, sem_b)
            reduce(2 * i + 1, buf_b)
            return carry

        lax.fori_loop(0, RPW // 2, pair_body, 0)
        pltpu.sync_copy(res_v, out_hbm.at[pl.ds(wid * RPW, RPW)])

    return k(table, idx2d)


def _tc_head(x, W, b, gamma, beta):
    def body(x_ref, w_ref, b_ref, g_ref, be_ref, o_ref):
        h = jnp.dot(x_ref[...], w_ref[...],
                    preferred_element_type=jnp.float32)
        h = h + b_ref[...]
        mean = jnp.mean(h, axis=-1, keepdims=True)
        c = h - mean
        var = jnp.mean(c * c, axis=-1, keepdims=True)
        hn = c * lax.rsqrt(var + 1e-5)
        hn = hn * g_ref[...] + be_ref[...]
        o_ref[...] = jnp.maximum(hn, 0.0)

    return pl.pallas_call(
        body,
        out_shape=jax.ShapeDtypeStruct((B, D_OUT), jnp.float32),
    )(x, W, b.reshape(1, D_OUT), gamma.reshape(1, D_OUT),
      beta.reshape(1, D_OUT))


def kernel(graph_embedding, locations_idx, W, b, gamma, beta):
    idx2d = locations_idx.astype(jnp.int32).reshape(B * NCH, CH)
    x = _sc_gather_max(graph_embedding, idx2d)
    return _tc_head(x, W, b, gamma, beta)
